# baseline (speedup 1.0000x reference)
import functools

import jax
import jax.numpy as jnp
from jax import lax
from jax.experimental import pallas as pl
from jax.experimental.pallas import tpu as pltpu
from jax.experimental.pallas import tpu_sc as plsc

NC, NS = 2, 16
NW = NC * NS
CHUNK = 128
SPLIT0 = 0.75


def _degree_kernel(n_pad, ept):
    mesh = plsc.VectorSubcoreMesh(core_axis_name="c", subcore_axis_name="s", num_cores=NC, num_subcores=NS)

    @functools.partial(
        pl.kernel,
        out_type=jax.ShapeDtypeStruct((NW, n_pad), jnp.float32),
        mesh=mesh,
        compiler_params=pltpu.CompilerParams(needs_layout_passes=False),
        scratch_types=[
            pltpu.VMEM((ept,), jnp.int32),
            pltpu.VMEM((n_pad,), jnp.float32),
        ],
    )
    def k(dst_flat, out, idx_v, acc):
        cid = lax.axis_index("c")
        sid = lax.axis_index("s")
        wid = cid * NS + sid
        zeros = jnp.zeros((16,), jnp.float32)

        def zbody(i, carry):
            acc[pl.ds(16 * i, 16)] = zeros
            return carry

        lax.fori_loop(0, n_pad // 16, zbody, 0)
        pltpu.sync_copy(dst_flat.at[pl.ds(wid * ept, ept)], idx_v)
        ones = jnp.ones((16,), jnp.float32)

        def body(i, carry):
            iv = idx_v[pl.ds(16 * i, 16)]
            plsc.addupdate_scatter(acc, [iv], ones)
            return carry

        lax.fori_loop(0, ept // 16, body, 0)
        pltpu.sync_copy(acc, out.at[wid])

    return k


def _propagate_kernel(n_pad, width, nc0, nc1):
    rows_per_tile = n_pad // NS
    mesh = plsc.VectorSubcoreMesh(core_axis_name="c", subcore_axis_name="s", num_cores=NC, num_subcores=NS)

    NB = 3

    @functools.partial(
        pl.kernel,
        out_type=jax.ShapeDtypeStruct((NC, n_pad, width), jnp.float32),
        mesh=mesh,
        scratch_types=[
            pltpu.VMEM((NB, 2, CHUNK), jnp.int32),
            pltpu.VMEM((NB, CHUNK, width), jnp.float32),
            pltpu.VMEM_SHARED((n_pad, width), jnp.float32),
            pltpu.SemaphoreType.DMA,
            pltpu.SemaphoreType.DMA,
        ],
    )
    def k(u_hbm, sdb, zeros_hbm, out, idxr, rb, acc, gsem, ssem):
        cid = lax.axis_index("c")
        sid = lax.axis_index("s")
        r0 = sid * rows_per_tile
        pltpu.sync_copy(zeros_hbm.at[pl.ds(r0, rows_per_tile)],
                        acc.at[pl.ds(r0, rows_per_tile)])
        plsc.subcore_barrier()

        def run(base, n_chunks):
            def fetch_idx(j):
                pltpu.sync_copy(sdb.at[base + j], idxr.at[j % NB])

            def fire_gather(j):
                pltpu.async_copy(u_hbm.at[idxr.at[j % NB, 0]], rb.at[j % NB],
                                 gsem)

            def wait_gather(b):
                pltpu.make_async_copy(u_hbm.at[idxr.at[0, 0]], rb.at[b],
                                      gsem).wait()

            def wait_scatter():
                pltpu.make_async_copy(rb.at[0], acc.at[idxr.at[0, 1]],
                                      ssem).wait()

            fetch_idx(0)
            fire_gather(0)
            fetch_idx(1)
            fire_gather(1)
            n_drained = 0
            for j in range(n_chunks):
                b = j % NB
                wait_gather(b)
                pltpu.async_copy(rb.at[b], acc.at[idxr.at[b, 1]], ssem,
                                 add=True)
                if j + 2 < n_chunks:
                    if j - 1 >= 0:
                        wait_scatter()
                        n_drained += 1
                    fetch_idx(j + 2)
                    fire_gather(j + 2)
            for _ in range(n_chunks - n_drained):
                wait_scatter()

        if nc0 == nc1:
            run(lax.axis_index("c") * NS * nc0 + sid * nc0, nc0)
        else:
            @pl.when(cid == 0)
            def _():
                run(sid * nc0, nc0)

            @pl.when(cid == 1)
            def _():
                run(NS * nc0 + sid * nc1, nc1)

        plsc.subcore_barrier()
        pltpu.sync_copy(acc.at[pl.ds(r0, rows_per_tile)],
                        out.at[cid, pl.ds(r0, rows_per_tile)])

    return k


def _prep_body(dacct_ref, x_ref, dinv_ref, u1_ref):
    deg = 1.0 + jnp.sum(dacct_ref[...], axis=1, keepdims=True)
    dinv = lax.rsqrt(deg)
    dinvb = jnp.broadcast_to(dinv, x_ref.shape)
    dinv_ref[...] = dinvb
    u1_ref[...] = x_ref[...] * dinvb


def _mid_body(sacc_ref, x_ref, dinv_ref, W1_ref, b1_ref, W2_ref, a1_ref,
              t_ref, u2_ref):
    dinv = dinv_ref[...]
    z1 = dinv * (sacc_ref[0] + sacc_ref[1]) + dinv * dinv * x_ref[...]
    h = jnp.dot(z1, W1_ref[...], preferred_element_type=jnp.float32) + b1_ref[...]
    a1 = a1_ref[0, 0]
    h = jnp.where(h >= 0, h, a1 * h)
    t = jnp.dot(h, W2_ref[...], preferred_element_type=jnp.float32)
    t_ref[...] = t
    u2_ref[...] = t * dinv


def _final_body(sacc_ref, t_ref, dinv_ref, b2_ref, a2_ref, out_ref):
    dinv = dinv_ref[...]
    z = dinv * (sacc_ref[0] + sacc_ref[1]) + dinv * dinv * t_ref[...] + b2_ref[...]
    a2 = a2_ref[0, 0]
    out_ref[...] = jnp.where(z >= 0, z, a2 * z)


def kernel(x, edge_index, W1, b1, W2, b2, a1, a2):
    n, d = x.shape
    o = W2.shape[1]
    src = edge_index[0].astype(jnp.int32)
    dst = edge_index[1].astype(jnp.int32)
    e = src.shape[0]

    n_pad = ((n + 1 + 127) // 128) * 128
    n_chunks = -(-e // (NW * CHUNK))
    n_chunks = ((n_chunks + 7) // 8) * 8
    e_pad = NW * n_chunks * CHUNK
    pad = e_pad - e
    srcp = jnp.concatenate([src, jnp.full((pad,), n, jnp.int32)]
                           ).reshape(NW * n_chunks, CHUNK)
    dstp = jnp.concatenate([dst, jnp.full((pad,), n, jnp.int32)]
                           ).reshape(NW * n_chunks, CHUNK)
    sdp = jnp.stack([srcp, dstp], axis=1)
    zerosw = jnp.zeros((n_pad, d), jnp.float32)
    xp = jnp.pad(x, ((0, n_pad - n), (0, 0)))

    r = 632
    grid = (n_pad // r,)
    row_spec = pl.BlockSpec((r, d), lambda i: (i, 0))
    acc_spec = pl.BlockSpec((NC, r, d), lambda i: (0, i, 0))
    smem_spec = pl.BlockSpec(memory_space=pltpu.SMEM)

    ept = n_chunks * CHUNK
    dacc = _degree_kernel(n_pad, ept)(dstp.reshape(-1))

    dinv128, u1p = pl.pallas_call(
        _prep_body,
        grid=grid,
        in_specs=[pl.BlockSpec((r, NW), lambda i: (i, 0)), row_spec],
        out_specs=[row_spec, row_spec],
        out_shape=[jax.ShapeDtypeStruct((n_pad, d), jnp.float32)] * 2,
    )(dacc.T, xp)

    nc_pair = 2 * n_chunks
    nc0 = int(round(nc_pair * SPLIT0))
    nc1 = nc_pair - nc0
    s1 = _propagate_kernel(n_pad, d, nc0, nc1)(u1p, sdp, zerosw)

    t, u2p = pl.pallas_call(
        _mid_body,
        grid=grid,
        in_specs=[acc_spec, row_spec, row_spec,
                  pl.BlockSpec(W1.shape, lambda i: (0, 0)),
                  pl.BlockSpec((1, W1.shape[1]), lambda i: (0, 0)),
                  pl.BlockSpec(W2.shape, lambda i: (0, 0)),
                  smem_spec],
        out_specs=[pl.BlockSpec((r, o), lambda i: (i, 0))] * 2,
        out_shape=[jax.ShapeDtypeStruct((n_pad, o), jnp.float32)] * 2,
    )(s1, xp, dinv128, W1, b1.reshape(1, -1), W2, a1.reshape(1, 1))

    s2 = _propagate_kernel(n_pad, o, nc0, nc1)(u2p, sdp, zerosw)

    out = pl.pallas_call(
        _final_body,
        grid=grid,
        in_specs=[pl.BlockSpec((NC, r, o), lambda i: (0, i, 0)),
                  pl.BlockSpec((r, o), lambda i: (i, 0)), row_spec,
                  pl.BlockSpec((1, o), lambda i: (0, 0)), smem_spec],
        out_specs=pl.BlockSpec((r, o), lambda i: (i, 0)),
        out_shape=jax.ShapeDtypeStruct((n_pad, o), jnp.float32),
    )(s2, t, dinv128, b2.reshape(1, -1), a2.reshape(1, 1))

    return out[:n]


# speedup vs baseline: 1.0329x; 1.0329x over previous
import functools

import jax
import jax.numpy as jnp
from jax import lax
from jax.experimental import pallas as pl
from jax.experimental.pallas import tpu as pltpu
from jax.experimental.pallas import tpu_sc as plsc

NC, NS = 2, 16
NW = NC * NS
CHUNK = 128
SPLIT0 = 0.5


def _degree_kernel(n_pad, ept):
    mesh = plsc.VectorSubcoreMesh(core_axis_name="c", subcore_axis_name="s", num_cores=NC, num_subcores=NS)

    @functools.partial(
        pl.kernel,
        out_type=jax.ShapeDtypeStruct((NW, n_pad), jnp.float32),
        mesh=mesh,
        compiler_params=pltpu.CompilerParams(needs_layout_passes=False),
        scratch_types=[
            pltpu.VMEM((ept,), jnp.int32),
            pltpu.VMEM((n_pad,), jnp.float32),
        ],
    )
    def k(dst_flat, out, idx_v, acc):
        cid = lax.axis_index("c")
        sid = lax.axis_index("s")
        wid = cid * NS + sid
        zeros = jnp.zeros((16,), jnp.float32)

        def zbody(i, carry):
            acc[pl.ds(16 * i, 16)] = zeros
            return carry

        lax.fori_loop(0, n_pad // 16, zbody, 0)
        pltpu.sync_copy(dst_flat.at[pl.ds(wid * ept, ept)], idx_v)
        ones = jnp.ones((16,), jnp.float32)

        def body(i, carry):
            iv = idx_v[pl.ds(16 * i, 16)]
            plsc.addupdate_scatter(acc, [iv], ones)
            return carry

        lax.fori_loop(0, ept // 16, body, 0)
        pltpu.sync_copy(acc, out.at[wid])

    return k


def _propagate_kernel(n_pad, width, nc0, nc1):
    rows_per_tile = n_pad // NS
    mesh = plsc.VectorSubcoreMesh(core_axis_name="c", subcore_axis_name="s", num_cores=NC, num_subcores=NS)

    NB = 3

    @functools.partial(
        pl.kernel,
        out_type=jax.ShapeDtypeStruct((NC, n_pad, width), jnp.float32),
        mesh=mesh,
        scratch_types=[
            pltpu.VMEM((NB, 2, CHUNK), jnp.int32),
            pltpu.VMEM((NB, CHUNK, width), jnp.float32),
            pltpu.VMEM_SHARED((n_pad, width), jnp.float32),
            pltpu.SemaphoreType.DMA,
            pltpu.SemaphoreType.DMA,
        ],
    )
    def k(u_hbm, sdb, out, idxr, rb, acc, gsem, ssem):
        cid = lax.axis_index("c")
        sid = lax.axis_index("s")
        r0 = sid * rows_per_tile
        zeros = jnp.zeros((16,), jnp.float32)

        def zbody(i, carry):
            rb[0, i >> 3, pl.ds((i & 7) * 16, 16)] = zeros
            return carry

        lax.fori_loop(0, CHUNK * (width // 16), zbody, 0)
        z0 = 0
        while z0 < rows_per_tile:
            step = min(CHUNK, rows_per_tile - z0)
            pltpu.sync_copy(rb.at[0, pl.ds(0, step)],
                            acc.at[pl.ds(r0 + z0, step)])
            z0 += step
        plsc.subcore_barrier()

        def run(base, n_chunks):
            def fetch_idx(j):
                pltpu.sync_copy(sdb.at[base + j], idxr.at[j % NB])

            def fire_gather(j):
                pltpu.async_copy(u_hbm.at[idxr.at[j % NB, 0]], rb.at[j % NB],
                                 gsem)

            def wait_gather(b):
                pltpu.make_async_copy(u_hbm.at[idxr.at[0, 0]], rb.at[b],
                                      gsem).wait()

            def wait_scatter():
                pltpu.make_async_copy(rb.at[0], acc.at[idxr.at[0, 1]],
                                      ssem).wait()

            fetch_idx(0)
            fire_gather(0)
            fetch_idx(1)
            fire_gather(1)
            n_drained = 0
            for j in range(n_chunks):
                b = j % NB
                wait_gather(b)
                pltpu.async_copy(rb.at[b], acc.at[idxr.at[b, 1]], ssem,
                                 add=True)
                if j + 2 < n_chunks:
                    if j - 1 >= 0:
                        wait_scatter()
                        n_drained += 1
                    fetch_idx(j + 2)
                    fire_gather(j + 2)
            for _ in range(n_chunks - n_drained):
                wait_scatter()

        if nc0 == nc1:
            run(lax.axis_index("c") * NS * nc0 + sid * nc0, nc0)
        else:
            @pl.when(cid == 0)
            def _():
                run(sid * nc0, nc0)

            @pl.when(cid == 1)
            def _():
                run(NS * nc0 + sid * nc1, nc1)

        plsc.subcore_barrier()
        pltpu.sync_copy(acc.at[pl.ds(r0, rows_per_tile)],
                        out.at[cid, pl.ds(r0, rows_per_tile)])

    return k


def _prep_body(dacct_ref, x_ref, dinv_ref, u1_ref):
    deg = 1.0 + jnp.sum(dacct_ref[...], axis=1, keepdims=True)
    dinv = lax.rsqrt(deg)
    dinvb = jnp.broadcast_to(dinv, x_ref.shape)
    dinv_ref[...] = dinvb
    u1_ref[...] = x_ref[...] * dinvb


def _mid_body(sacc_ref, x_ref, dinv_ref, W1_ref, b1_ref, W2_ref, a1_ref,
              t_ref, u2_ref):
    dinv = dinv_ref[...]
    z1 = dinv * (sacc_ref[0] + sacc_ref[1]) + dinv * dinv * x_ref[...]
    h = jnp.dot(z1, W1_ref[...], preferred_element_type=jnp.float32) + b1_ref[...]
    a1 = a1_ref[0, 0]
    h = jnp.where(h >= 0, h, a1 * h)
    t = jnp.dot(h, W2_ref[...], preferred_element_type=jnp.float32)
    t_ref[...] = t
    u2_ref[...] = t * dinv


def _final_body(sacc_ref, t_ref, dinv_ref, b2_ref, a2_ref, out_ref):
    dinv = dinv_ref[...]
    z = dinv * (sacc_ref[0] + sacc_ref[1]) + dinv * dinv * t_ref[...] + b2_ref[...]
    a2 = a2_ref[0, 0]
    out_ref[...] = jnp.where(z >= 0, z, a2 * z)


def kernel(x, edge_index, W1, b1, W2, b2, a1, a2):
    n, d = x.shape
    o = W2.shape[1]
    src = edge_index[0].astype(jnp.int32)
    dst = edge_index[1].astype(jnp.int32)
    e = src.shape[0]

    n_pad = ((n + 1 + 127) // 128) * 128
    n_chunks = -(-e // (NW * CHUNK))
    n_chunks = ((n_chunks + 7) // 8) * 8
    e_pad = NW * n_chunks * CHUNK
    pad = e_pad - e
    srcp = jnp.concatenate([src, jnp.full((pad,), n, jnp.int32)]
                           ).reshape(NW * n_chunks, CHUNK)
    dstp = jnp.concatenate([dst, jnp.full((pad,), n, jnp.int32)]
                           ).reshape(NW * n_chunks, CHUNK)
    sdp = jnp.stack([srcp, dstp], axis=1)
    xp = jnp.pad(x, ((0, n_pad - n), (0, 0)))

    r = 632
    grid = (n_pad // r,)
    row_spec = pl.BlockSpec((r, d), lambda i: (i, 0))
    acc_spec = pl.BlockSpec((NC, r, d), lambda i: (0, i, 0))
    smem_spec = pl.BlockSpec(memory_space=pltpu.SMEM)

    ept = n_chunks * CHUNK
    dacc = _degree_kernel(n_pad, ept)(dstp.reshape(-1))

    dinv128, u1p = pl.pallas_call(
        _prep_body,
        grid=grid,
        in_specs=[pl.BlockSpec((r, NW), lambda i: (i, 0)), row_spec],
        out_specs=[row_spec, row_spec],
        out_shape=[jax.ShapeDtypeStruct((n_pad, d), jnp.float32)] * 2,
    )(dacc.T, xp)

    nc_pair = 2 * n_chunks
    nc0 = int(round(nc_pair * SPLIT0))
    nc1 = nc_pair - nc0
    s1 = _propagate_kernel(n_pad, d, nc0, nc1)(u1p, sdp)

    t, u2p = pl.pallas_call(
        _mid_body,
        grid=grid,
        in_specs=[acc_spec, row_spec, row_spec,
                  pl.BlockSpec(W1.shape, lambda i: (0, 0)),
                  pl.BlockSpec((1, W1.shape[1]), lambda i: (0, 0)),
                  pl.BlockSpec(W2.shape, lambda i: (0, 0)),
                  smem_spec],
        out_specs=[pl.BlockSpec((r, o), lambda i: (i, 0))] * 2,
        out_shape=[jax.ShapeDtypeStruct((n_pad, o), jnp.float32)] * 2,
    )(s1, xp, dinv128, W1, b1.reshape(1, -1), W2, a1.reshape(1, 1))

    s2 = _propagate_kernel(n_pad, o, nc0, nc1)(u2p, sdp)

    out = pl.pallas_call(
        _final_body,
        grid=grid,
        in_specs=[pl.BlockSpec((NC, r, o), lambda i: (0, i, 0)),
                  pl.BlockSpec((r, o), lambda i: (i, 0)), row_spec,
                  pl.BlockSpec((1, o), lambda i: (0, 0)), smem_spec],
        out_specs=pl.BlockSpec((r, o), lambda i: (i, 0)),
        out_shape=jax.ShapeDtypeStruct((n_pad, o), jnp.float32),
    )(s2, t, dinv128, b2.reshape(1, -1), a2.reshape(1, 1))

    return out[:n]
